# baseline (speedup 1.0000x reference)
import math

import jax
import jax.numpy as jnp
from jax.experimental import pallas as pl

G = 512
A = 16
N = G * A
TYPE_DIM = 100
TIME_DIM = 128
H = 128
L = 4
NFREQ = 10
GB = 32


def _silu(x):
    t = jnp.tanh((0.5 * x).astype(jnp.bfloat16))
    return x * (0.5 * t.astype(jnp.float32) + 0.5)


def _ln(x, g, b):
    m = jnp.mean(x, axis=-1, keepdims=True)
    xc = x - m
    v = jnp.mean(xc * xc, axis=-1, keepdims=True)
    return xc * jax.lax.rsqrt(v + 1e-5) * g + b


def _dot(x, w):
    return jax.lax.dot_general(
        x, w, (((x.ndim - 1,), (0,)), ((), ())),
        preferred_element_type=jnp.float32)


def _gnn_block(t_ref, at_ref, fc_ref, lp_ref, tfreq_ref, sfreq_ref,
               WtsT_ref, bts_ref, WneA_ref, WneB_ref, bne_ref,
               mW1a_ref, mW1b_ref, mW1s_ref, mW1c_ref, mW1lp_ref, mb1_ref,
               mW2_ref, mb2_ref,
               aW1a_ref, aW1b_ref, ab1_ref, aW2_ref, ab2_ref,
               lng_ref, lnb_ref, flng_ref, flnb_ref,
               WtrT_ref, btr_ref, WlpT_ref, WfcT_ref,
               type_out, lpp_out, fcp_out):
    nb = GB * A
    E = GB * A * A
    inv_a = 1.0 / A

    temb = _dot(at_ref[...], WtsT_ref[...]) + bts_ref[...]
    targ = t_ref[...] * tfreq_ref[...]
    temb_t = jnp.concatenate([jnp.sin(targ), jnp.cos(targ)], axis=1)
    tproj = _dot(temb_t, WneB_ref[...])
    nf = _dot(temb, WneA_ref[...]) + bne_ref[...]
    nf = (nf.reshape(GB, A, H) + tproj[:, None, :]).reshape(nb, H)

    narg = _dot(fc_ref[...], sfreq_ref[...])
    ns = jnp.sin(narg)
    nc = jnp.cos(narg)
    nsj = ns.reshape(GB, 1, A, 3 * NFREQ)
    ncj = nc.reshape(GB, 1, A, 3 * NFREQ)
    nsi = ns.reshape(GB, A, 1, 3 * NFREQ)
    nci = nc.reshape(GB, A, 1, 3 * NFREQ)
    fsin = (nsj * nci - ncj * nsi).reshape(E, 3 * NFREQ)
    fcos = (ncj * nci + nsj * nsi).reshape(E, 3 * NFREQ)
    lp = lp_ref[...]

    for l in range(L):
        nfn = _ln(nf, lng_ref[l], lnb_ref[l])
        asrc = _dot(nfn, mW1a_ref[l])
        bdst = _dot(nfn, mW1b_ref[l])
        cef = _dot(fsin, mW1s_ref[l]) + _dot(fcos, mW1c_ref[l])
        lpp = _dot(lp, mW1lp_ref[l]) + mb1_ref[l]
        pre = (cef.reshape(GB, A, A, H)
               + asrc.reshape(GB, A, 1, H)
               + bdst.reshape(GB, 1, A, H)
               + lpp.reshape(GB, 1, 1, H))
        h = _dot(_silu(pre).reshape(E, H), mW2_ref[l]) + mb2_ref[l]
        mij = _silu(h)
        msg = jnp.sum(mij.reshape(nb, A, H), axis=1) * inv_a
        agg = _dot(nf, aW1a_ref[l]) + _dot(msg, aW1b_ref[l]) + ab1_ref[l]
        agg = _silu(_dot(_silu(agg), aW2_ref[l]) + ab2_ref[l])
        nf = nf + agg

    nff = _ln(nf, flng_ref[...], flnb_ref[...])
    gf = jnp.sum(nff.reshape(GB, A, H), axis=1) * inv_a
    type_out[...] = _dot(nff, WtrT_ref[...]) + btr_ref[...]
    lpp_out[...] = _dot(gf, WlpT_ref[...])
    fcp_out[...] = _dot(nff, WfcT_ref[...])


def kernel(t, num_atoms, atom_types, frac_coords, l_polar, node2graph,
           W_ts, b_ts, W_ne, b_ne, msg_W1, msg_b1, msg_W2, msg_b2,
           agg_W1, agg_b1, agg_W2, agg_b2, ln_g, ln_b, fln_g, fln_b,
           W_tr, b_tr, W_lp, W_fc):
    f32 = jnp.float32
    half = TIME_DIM // 2
    tfreq = jnp.exp(
        jnp.arange(half, dtype=f32) * (-(math.log(10000.0) / (half - 1)))
    ).reshape(1, half)
    sfreq = jnp.kron(
        jnp.eye(3, dtype=f32),
        (2.0 * math.pi * jnp.arange(NFREQ, dtype=f32)).reshape(1, NFREQ))
    t2 = t.reshape(G, 1)

    WtsT = W_ts.T
    bts = b_ts.reshape(1, H)
    WneA = W_ne[:, :H].T
    WneB = W_ne[:, H:].T
    bne = b_ne.reshape(1, H)
    tr = lambda w: jnp.transpose(w, (0, 2, 1))
    mW1a = tr(msg_W1[:, :, :H])
    mW1b = tr(msg_W1[:, :, H:2 * H])
    mW1s = tr(msg_W1[:, :, 2 * H:2 * H + 3 * NFREQ])
    mW1c = tr(msg_W1[:, :, 2 * H + 3 * NFREQ:2 * H + 6 * NFREQ])
    mW1lp = tr(msg_W1[:, :, 2 * H + 6 * NFREQ:])
    mb1 = msg_b1.reshape(L, 1, H)
    mW2 = tr(msg_W2)
    mb2 = msg_b2.reshape(L, 1, H)
    aW1a = tr(agg_W1[:, :, :H])
    aW1b = tr(agg_W1[:, :, H:])
    ab1 = agg_b1.reshape(L, 1, H)
    aW2 = tr(agg_W2)
    ab2 = agg_b2.reshape(L, 1, H)
    lng = ln_g.reshape(L, 1, H)
    lnb = ln_b.reshape(L, 1, H)
    flng = fln_g.reshape(1, H)
    flnb = fln_b.reshape(1, H)
    WtrT = W_tr.T
    btr = b_tr.reshape(1, TYPE_DIM)
    WlpT = W_lp.T
    WfcT = W_fc.T

    nb = GB * A

    def full(shape):
        return pl.BlockSpec(shape, lambda i: tuple(0 for _ in shape))

    def node_bs(d):
        return pl.BlockSpec((nb, d), lambda i: (i, 0))

    def graph_bs(d):
        return pl.BlockSpec((GB, d), lambda i: (i, 0))

    out_shapes = (
        jax.ShapeDtypeStruct((N, TYPE_DIM), f32),
        jax.ShapeDtypeStruct((G, 6), f32),
        jax.ShapeDtypeStruct((N, 3), f32),
    )
    out_specs = (node_bs(TYPE_DIM), graph_bs(6), node_bs(3))

    return pl.pallas_call(
        _gnn_block,
        grid=(G // GB,),
        in_specs=[graph_bs(1), node_bs(TYPE_DIM), node_bs(3), graph_bs(6),
                  full((1, half)), full((3, 3 * NFREQ)),
                  full((TYPE_DIM, H)), full((1, H)), full((H, H)),
                  full((H, H)), full((1, H)),
                  full((L, H, H)), full((L, H, H)), full((L, 3 * NFREQ, H)),
                  full((L, 3 * NFREQ, H)), full((L, 6, H)), full((L, 1, H)),
                  full((L, H, H)), full((L, 1, H)),
                  full((L, H, H)), full((L, H, H)), full((L, 1, H)),
                  full((L, H, H)), full((L, 1, H)),
                  full((L, 1, H)), full((L, 1, H)), full((1, H)),
                  full((1, H)),
                  full((H, TYPE_DIM)), full((1, TYPE_DIM)), full((H, 6)),
                  full((H, 3))],
        out_specs=out_specs,
        out_shape=out_shapes,
    )(t2, atom_types, frac_coords, l_polar, tfreq, sfreq,
      WtsT, bts, WneA, WneB, bne,
      mW1a, mW1b, mW1s, mW1c, mW1lp, mb1, mW2, mb2,
      aW1a, aW1b, ab1, aW2, ab2,
      lng, lnb, flng, flnb,
      WtrT, btr, WlpT, WfcT)


# speedup vs baseline: 1.0798x; 1.0798x over previous
import math

import jax
import jax.numpy as jnp
from jax.experimental import pallas as pl

G = 512
A = 16
N = G * A
TYPE_DIM = 100
TIME_DIM = 128
H = 128
L = 4
NFREQ = 10
GB = 32


def _silu(x):
    return x * (0.5 * jnp.tanh(0.5 * x) + 0.5)


def _ln(x, g, b):
    m = jnp.mean(x, axis=-1, keepdims=True)
    xc = x - m
    v = jnp.mean(xc * xc, axis=-1, keepdims=True)
    return xc * jax.lax.rsqrt(v + 1e-5) * g + b


def _dot(x, w):
    return jax.lax.dot_general(
        x, w, (((x.ndim - 1,), (0,)), ((), ())),
        preferred_element_type=jnp.float32)


def _gnn_block(t_ref, at_ref, fc_ref, lp_ref, tfreq_ref, sfreq_ref,
               WtsT_ref, bts_ref, WneA_ref, WneB_ref, bne_ref,
               mW1a_ref, mW1b_ref, mW1s_ref, mW1c_ref, mW1lp_ref, mb1_ref,
               mW2_ref, mb2_ref,
               aW1a_ref, aW1b_ref, ab1_ref, aW2_ref, ab2_ref,
               lng_ref, lnb_ref, flng_ref, flnb_ref,
               WtrT_ref, btr_ref, WlpT_ref, WfcT_ref,
               type_out, lpp_out, fcp_out):
    nb = GB * A
    E = GB * A * A
    inv_a = 1.0 / A

    temb = _dot(at_ref[...], WtsT_ref[...]) + bts_ref[...]
    targ = t_ref[...] * tfreq_ref[...]
    temb_t = jnp.concatenate([jnp.sin(targ), jnp.cos(targ)], axis=1)
    tproj = _dot(temb_t, WneB_ref[...])
    nf = _dot(temb, WneA_ref[...]) + bne_ref[...]
    nf = (nf.reshape(GB, A, H) + tproj[:, None, :]).reshape(nb, H)

    narg = _dot(fc_ref[...], sfreq_ref[...])
    ns = jnp.sin(narg)
    nc = jnp.cos(narg)
    nsj = ns.reshape(GB, 1, A, 3 * NFREQ)
    ncj = nc.reshape(GB, 1, A, 3 * NFREQ)
    nsi = ns.reshape(GB, A, 1, 3 * NFREQ)
    nci = nc.reshape(GB, A, 1, 3 * NFREQ)
    fsin = (nsj * nci - ncj * nsi).reshape(E, 3 * NFREQ)
    fcos = (ncj * nci + nsj * nsi).reshape(E, 3 * NFREQ)
    lp = lp_ref[...]

    for l in range(L):
        nfn = _ln(nf, lng_ref[l], lnb_ref[l])
        asrc = _dot(nfn, mW1a_ref[l])
        bdst = _dot(nfn, mW1b_ref[l])
        cef = _dot(fsin, mW1s_ref[l]) + _dot(fcos, mW1c_ref[l])
        lpp = _dot(lp, mW1lp_ref[l]) + mb1_ref[l]
        asrc = (asrc.reshape(GB, A, H) + lpp[:, None, :]).reshape(nb, H)
        pre = (cef.reshape(GB, A, A, H)
               + asrc.reshape(GB, A, 1, H)
               + bdst.reshape(GB, 1, A, H))
        h = _dot(_silu(pre).reshape(E, H), mW2_ref[l]) + mb2_ref[l]
        mij = _silu(h)
        msg = jnp.sum(mij.reshape(nb, A, H), axis=1) * inv_a
        agg = _dot(nf, aW1a_ref[l]) + _dot(msg, aW1b_ref[l]) + ab1_ref[l]
        agg = _silu(_dot(_silu(agg), aW2_ref[l]) + ab2_ref[l])
        nf = nf + agg

    nff = _ln(nf, flng_ref[...], flnb_ref[...])
    gf = jnp.sum(nff.reshape(GB, A, H), axis=1) * inv_a
    type_out[...] = _dot(nff, WtrT_ref[...]) + btr_ref[...]
    lpp_out[...] = _dot(gf, WlpT_ref[...])
    fcp_out[...] = _dot(nff, WfcT_ref[...])


def kernel(t, num_atoms, atom_types, frac_coords, l_polar, node2graph,
           W_ts, b_ts, W_ne, b_ne, msg_W1, msg_b1, msg_W2, msg_b2,
           agg_W1, agg_b1, agg_W2, agg_b2, ln_g, ln_b, fln_g, fln_b,
           W_tr, b_tr, W_lp, W_fc):
    f32 = jnp.float32
    half = TIME_DIM // 2
    tfreq = jnp.exp(
        jnp.arange(half, dtype=f32) * (-(math.log(10000.0) / (half - 1)))
    ).reshape(1, half)
    sfreq = jnp.kron(
        jnp.eye(3, dtype=f32),
        (2.0 * math.pi * jnp.arange(NFREQ, dtype=f32)).reshape(1, NFREQ))
    t2 = t.reshape(G, 1)

    WtsT = W_ts.T
    bts = b_ts.reshape(1, H)
    WneA = W_ne[:, :H].T
    WneB = W_ne[:, H:].T
    bne = b_ne.reshape(1, H)
    tr = lambda w: jnp.transpose(w, (0, 2, 1))
    mW1a = tr(msg_W1[:, :, :H])
    mW1b = tr(msg_W1[:, :, H:2 * H])
    mW1s = tr(msg_W1[:, :, 2 * H:2 * H + 3 * NFREQ])
    mW1c = tr(msg_W1[:, :, 2 * H + 3 * NFREQ:2 * H + 6 * NFREQ])
    mW1lp = tr(msg_W1[:, :, 2 * H + 6 * NFREQ:])
    mb1 = msg_b1.reshape(L, 1, H)
    mW2 = tr(msg_W2)
    mb2 = msg_b2.reshape(L, 1, H)
    aW1a = tr(agg_W1[:, :, :H])
    aW1b = tr(agg_W1[:, :, H:])
    ab1 = agg_b1.reshape(L, 1, H)
    aW2 = tr(agg_W2)
    ab2 = agg_b2.reshape(L, 1, H)
    lng = ln_g.reshape(L, 1, H)
    lnb = ln_b.reshape(L, 1, H)
    flng = fln_g.reshape(1, H)
    flnb = fln_b.reshape(1, H)
    WtrT = W_tr.T
    btr = b_tr.reshape(1, TYPE_DIM)
    WlpT = W_lp.T
    WfcT = W_fc.T

    nb = GB * A

    def full(shape):
        return pl.BlockSpec(shape, lambda i: tuple(0 for _ in shape))

    def node_bs(d):
        return pl.BlockSpec((nb, d), lambda i: (i, 0))

    def graph_bs(d):
        return pl.BlockSpec((GB, d), lambda i: (i, 0))

    out_shapes = (
        jax.ShapeDtypeStruct((N, TYPE_DIM), f32),
        jax.ShapeDtypeStruct((G, 6), f32),
        jax.ShapeDtypeStruct((N, 3), f32),
    )
    out_specs = (node_bs(TYPE_DIM), graph_bs(6), node_bs(3))

    return pl.pallas_call(
        _gnn_block,
        grid=(G // GB,),
        in_specs=[graph_bs(1), node_bs(TYPE_DIM), node_bs(3), graph_bs(6),
                  full((1, half)), full((3, 3 * NFREQ)),
                  full((TYPE_DIM, H)), full((1, H)), full((H, H)),
                  full((H, H)), full((1, H)),
                  full((L, H, H)), full((L, H, H)), full((L, 3 * NFREQ, H)),
                  full((L, 3 * NFREQ, H)), full((L, 6, H)), full((L, 1, H)),
                  full((L, H, H)), full((L, 1, H)),
                  full((L, H, H)), full((L, H, H)), full((L, 1, H)),
                  full((L, H, H)), full((L, 1, H)),
                  full((L, 1, H)), full((L, 1, H)), full((1, H)),
                  full((1, H)),
                  full((H, TYPE_DIM)), full((1, TYPE_DIM)), full((H, 6)),
                  full((H, 3))],
        out_specs=out_specs,
        out_shape=out_shapes,
    )(t2, atom_types, frac_coords, l_polar, tfreq, sfreq,
      WtsT, bts, WneA, WneB, bne,
      mW1a, mW1b, mW1s, mW1c, mW1lp, mb1, mW2, mb2,
      aW1a, aW1b, ab1, aW2, ab2,
      lng, lnb, flng, flnb,
      WtrT, btr, WlpT, WfcT)


# speedup vs baseline: 1.1110x; 1.0289x over previous
import math

import jax
import jax.numpy as jnp
from jax.experimental import pallas as pl

G = 512
A = 16
N = G * A
TYPE_DIM = 100
TIME_DIM = 128
H = 128
L = 4
NFREQ = 10
GB = 32


def _silu(x):
    return x * (0.5 * jnp.tanh(0.5 * x) + 0.5)


def _ln(x, g, b):
    m = jnp.mean(x, axis=-1, keepdims=True)
    xc = x - m
    v = jnp.mean(xc * xc, axis=-1, keepdims=True)
    return xc * jax.lax.rsqrt(v + 1e-5) * g + b


def _dotT(x, w):
    return jax.lax.dot_general(
        x, w, (((1,), (1,)), ((), ())), preferred_element_type=jnp.float32)


def _gnn_block(t_ref, at_ref, fc_ref, lp_ref, tfreq_ref, sfreq_ref,
               Wts_ref, bts_ref, Wne_ref, bne_ref,
               mW1_ref, mb1_ref, mW2_ref, mb2_ref,
               aW1_ref, ab1_ref, aW2_ref, ab2_ref,
               lng_ref, lnb_ref, flng_ref, flnb_ref,
               Wtr_ref, btr_ref, Wlp_ref, Wfc_ref,
               type_out, lpp_out, fcp_out):
    nb = GB * A
    E = GB * A * A
    inv_a = 1.0 / A
    F3 = 3 * NFREQ

    temb = _dotT(at_ref[...], Wts_ref[...]) + bts_ref[...]
    targ = t_ref[...] * tfreq_ref[...]
    temb_t = jnp.concatenate([jnp.sin(targ), jnp.cos(targ)], axis=1)
    Wne = Wne_ref[...]
    tproj = _dotT(temb_t, Wne[:, H:])
    nf = _dotT(temb, Wne[:, :H]) + bne_ref[...]
    nf = (nf.reshape(GB, A, H) + tproj[:, None, :]).reshape(nb, H)

    narg = jax.lax.dot_general(
        fc_ref[...], sfreq_ref[...], (((1,), (0,)), ((), ())),
        preferred_element_type=jnp.float32)
    ns = jnp.sin(narg)
    nc = jnp.cos(narg)
    nsj = ns.reshape(GB, 1, A, F3)
    ncj = nc.reshape(GB, 1, A, F3)
    nsi = ns.reshape(GB, A, 1, F3)
    nci = nc.reshape(GB, A, 1, F3)
    fsin = (nsj * nci - ncj * nsi).reshape(E, F3)
    fcos = (ncj * nci + nsj * nsi).reshape(E, F3)
    lp = lp_ref[...]

    for l in range(L):
        w1 = mW1_ref[l]
        nfn = _ln(nf, lng_ref[l], lnb_ref[l])
        asrc = _dotT(nfn, w1[:, :H])
        bdst = _dotT(nfn, w1[:, H:2 * H])
        cef = (_dotT(fsin, w1[:, 2 * H:2 * H + F3])
               + _dotT(fcos, w1[:, 2 * H + F3:2 * H + 2 * F3]))
        lpp = _dotT(lp, w1[:, 2 * H + 2 * F3:]) + mb1_ref[l]
        asrc = (asrc.reshape(GB, A, H) + lpp[:, None, :]).reshape(nb, H)
        pre = (cef.reshape(GB, A, A, H)
               + asrc.reshape(GB, A, 1, H)
               + bdst.reshape(GB, 1, A, H))
        h = _dotT(_silu(pre).reshape(E, H), mW2_ref[l]) + mb2_ref[l]
        mij = _silu(h)
        msg = jnp.sum(mij.reshape(nb, A, H), axis=1) * inv_a
        aw1 = aW1_ref[l]
        agg = _dotT(nf, aw1[:, :H]) + _dotT(msg, aw1[:, H:]) + ab1_ref[l]
        agg = _silu(_dotT(_silu(agg), aW2_ref[l]) + ab2_ref[l])
        nf = nf + agg

    nff = _ln(nf, flng_ref[...], flnb_ref[...])
    gf = jnp.sum(nff.reshape(GB, A, H), axis=1) * inv_a
    type_out[...] = _dotT(nff, Wtr_ref[...]) + btr_ref[...]
    lpp_out[...] = _dotT(gf, Wlp_ref[...])
    fcp_out[...] = _dotT(nff, Wfc_ref[...])


def kernel(t, num_atoms, atom_types, frac_coords, l_polar, node2graph,
           W_ts, b_ts, W_ne, b_ne, msg_W1, msg_b1, msg_W2, msg_b2,
           agg_W1, agg_b1, agg_W2, agg_b2, ln_g, ln_b, fln_g, fln_b,
           W_tr, b_tr, W_lp, W_fc):
    f32 = jnp.float32
    half = TIME_DIM // 2
    msg_in = 2 * H + 6 * NFREQ + 6
    tfreq = jnp.exp(
        jnp.arange(half, dtype=f32) * (-(math.log(10000.0) / (half - 1)))
    ).reshape(1, half)
    sfreq = jnp.kron(
        jnp.eye(3, dtype=f32),
        (2.0 * math.pi * jnp.arange(NFREQ, dtype=f32)).reshape(1, NFREQ))
    t2 = t.reshape(G, 1)

    nb = GB * A

    def full(shape):
        return pl.BlockSpec(shape, lambda i: tuple(0 for _ in shape))

    def node_bs(d):
        return pl.BlockSpec((nb, d), lambda i: (i, 0))

    def graph_bs(d):
        return pl.BlockSpec((GB, d), lambda i: (i, 0))

    out_shapes = (
        jax.ShapeDtypeStruct((N, TYPE_DIM), f32),
        jax.ShapeDtypeStruct((G, 6), f32),
        jax.ShapeDtypeStruct((N, 3), f32),
    )
    out_specs = (node_bs(TYPE_DIM), graph_bs(6), node_bs(3))

    return pl.pallas_call(
        _gnn_block,
        grid=(G // GB,),
        in_specs=[graph_bs(1), node_bs(TYPE_DIM), node_bs(3), graph_bs(6),
                  full((1, half)), full((3, 3 * NFREQ)),
                  full((H, TYPE_DIM)), full((1, H)),
                  full((H, H + TIME_DIM)), full((1, H)),
                  full((L, H, msg_in)), full((L, 1, H)),
                  full((L, H, H)), full((L, 1, H)),
                  full((L, H, 2 * H)), full((L, 1, H)),
                  full((L, H, H)), full((L, 1, H)),
                  full((L, 1, H)), full((L, 1, H)), full((1, H)),
                  full((1, H)),
                  full((TYPE_DIM, H)), full((1, TYPE_DIM)), full((6, H)),
                  full((3, H))],
        out_specs=out_specs,
        out_shape=out_shapes,
    )(t2, atom_types, frac_coords, l_polar, tfreq, sfreq,
      W_ts, b_ts.reshape(1, H), W_ne, b_ne.reshape(1, H),
      msg_W1, msg_b1.reshape(L, 1, H), msg_W2, msg_b2.reshape(L, 1, H),
      agg_W1, agg_b1.reshape(L, 1, H), agg_W2, agg_b2.reshape(L, 1, H),
      ln_g.reshape(L, 1, H), ln_b.reshape(L, 1, H),
      fln_g.reshape(1, H), fln_b.reshape(1, H),
      W_tr, b_tr.reshape(1, TYPE_DIM), W_lp, W_fc)


# speedup vs baseline: 1.1594x; 1.0436x over previous
import math

import jax
import jax.numpy as jnp
from jax.experimental import pallas as pl

G = 512
A = 16
N = G * A
TYPE_DIM = 100
TIME_DIM = 128
H = 128
L = 4
NFREQ = 10
GB = 32


def _silu(x):
    return x * (0.5 * jnp.tanh(0.5 * x) + 0.5)


def _ln(x, g, b):
    m = jnp.mean(x, axis=-1, keepdims=True)
    xc = x - m
    v = jnp.mean(xc * xc, axis=-1, keepdims=True)
    return xc * jax.lax.rsqrt(v + 1e-5) * g + b


def _dotT(x, w):
    return jax.lax.dot_general(
        x, w, (((1,), (1,)), ((), ())), preferred_element_type=jnp.float32)


def _gnn_block(t_ref, at_ref, fc_ref, lp_ref, tfreq_ref, sfreq_ref,
               Wts_ref, bts_ref, Wne_ref, bne_ref,
               mW1_ref, mb1_ref, mW2_ref, mb2_ref,
               aW1_ref, ab1_ref, aW2_ref, ab2_ref,
               lng_ref, lnb_ref, flng_ref, flnb_ref,
               Wtr_ref, btr_ref, Wlp_ref, Wfc_ref,
               sred_ref,
               type_out, lpp_out, fcp_out):
    nb = GB * A
    E = GB * A * A
    inv_a = 1.0 / A
    F3 = 3 * NFREQ

    temb = _dotT(at_ref[...], Wts_ref[...]) + bts_ref[...]
    targ = t_ref[...] * tfreq_ref[...]
    temb_t = jnp.concatenate([jnp.sin(targ), jnp.cos(targ)], axis=1)
    Wne = Wne_ref[...]
    tproj = _dotT(temb_t, Wne[:, H:])
    nf = _dotT(temb, Wne[:, :H]) + bne_ref[...]
    nf = (nf.reshape(GB, A, H) + tproj[:, None, :]).reshape(nb, H)

    narg = jax.lax.dot_general(
        fc_ref[...], sfreq_ref[...], (((1,), (0,)), ((), ())),
        preferred_element_type=jnp.float32)
    ns = jnp.sin(narg)
    nc = jnp.cos(narg)
    nsj = ns.reshape(GB, 1, A, F3)
    ncj = nc.reshape(GB, 1, A, F3)
    nsi = ns.reshape(GB, A, 1, F3)
    nci = nc.reshape(GB, A, 1, F3)
    fsin = (nsj * nci - ncj * nsi).reshape(E, F3)
    fcos = (ncj * nci + nsj * nsi).reshape(E, F3)
    lp = lp_ref[...]

    for l in range(L):
        w1 = mW1_ref[l]
        nfn = _ln(nf, lng_ref[l], lnb_ref[l])
        asrc = _dotT(nfn, w1[:, :H])
        bdst = _dotT(nfn, w1[:, H:2 * H])
        cef = (_dotT(fsin, w1[:, 2 * H:2 * H + F3])
               + _dotT(fcos, w1[:, 2 * H + F3:2 * H + 2 * F3]))
        lpp = _dotT(lp, w1[:, 2 * H + 2 * F3:]) + mb1_ref[l]
        asrc = (asrc.reshape(GB, A, H) + lpp[:, None, :]).reshape(nb, H)
        pre = (cef.reshape(GB, A, A, H)
               + asrc.reshape(GB, A, 1, H)
               + bdst.reshape(GB, 1, A, H))
        h = _dotT(_silu(pre).reshape(E, H), mW2_ref[l]) + mb2_ref[l]
        mij = _silu(h)
        msg = jax.lax.dot_general(
            sred_ref[...], mij.reshape(GB, A * A, H),
            (((2,), (1,)), ((0,), (0,))),
            preferred_element_type=jnp.float32).reshape(nb, H)
        aw1 = aW1_ref[l]
        agg = _dotT(nf, aw1[:, :H]) + _dotT(msg, aw1[:, H:]) + ab1_ref[l]
        agg = _silu(_dotT(_silu(agg), aW2_ref[l]) + ab2_ref[l])
        nf = nf + agg

    nff = _ln(nf, flng_ref[...], flnb_ref[...])
    gf = jnp.sum(nff.reshape(GB, A, H), axis=1) * inv_a
    type_out[...] = _dotT(nff, Wtr_ref[...]) + btr_ref[...]
    lpp_out[...] = _dotT(gf, Wlp_ref[...])
    fcp_out[...] = _dotT(nff, Wfc_ref[...])


def kernel(t, num_atoms, atom_types, frac_coords, l_polar, node2graph,
           W_ts, b_ts, W_ne, b_ne, msg_W1, msg_b1, msg_W2, msg_b2,
           agg_W1, agg_b1, agg_W2, agg_b2, ln_g, ln_b, fln_g, fln_b,
           W_tr, b_tr, W_lp, W_fc):
    f32 = jnp.float32
    half = TIME_DIM // 2
    msg_in = 2 * H + 6 * NFREQ + 6
    tfreq = jnp.exp(
        jnp.arange(half, dtype=f32) * (-(math.log(10000.0) / (half - 1)))
    ).reshape(1, half)
    sfreq = jnp.kron(
        jnp.eye(3, dtype=f32),
        (2.0 * math.pi * jnp.arange(NFREQ, dtype=f32)).reshape(1, NFREQ))
    t2 = t.reshape(G, 1)
    sred = jnp.broadcast_to(
        jnp.kron(jnp.eye(A, dtype=f32), jnp.full((1, A), 1.0 / A, f32)),
        (GB, A, A * A))

    nb = GB * A

    def full(shape):
        return pl.BlockSpec(shape, lambda i: tuple(0 for _ in shape))

    def node_bs(d):
        return pl.BlockSpec((nb, d), lambda i: (i, 0))

    def graph_bs(d):
        return pl.BlockSpec((GB, d), lambda i: (i, 0))

    out_shapes = (
        jax.ShapeDtypeStruct((N, TYPE_DIM), f32),
        jax.ShapeDtypeStruct((G, 6), f32),
        jax.ShapeDtypeStruct((N, 3), f32),
    )
    out_specs = (node_bs(TYPE_DIM), graph_bs(6), node_bs(3))

    return pl.pallas_call(
        _gnn_block,
        grid=(G // GB,),
        in_specs=[graph_bs(1), node_bs(TYPE_DIM), node_bs(3), graph_bs(6),
                  full((1, half)), full((3, 3 * NFREQ)),
                  full((H, TYPE_DIM)), full((1, H)),
                  full((H, H + TIME_DIM)), full((1, H)),
                  full((L, H, msg_in)), full((L, 1, H)),
                  full((L, H, H)), full((L, 1, H)),
                  full((L, H, 2 * H)), full((L, 1, H)),
                  full((L, H, H)), full((L, 1, H)),
                  full((L, 1, H)), full((L, 1, H)), full((1, H)),
                  full((1, H)),
                  full((TYPE_DIM, H)), full((1, TYPE_DIM)), full((6, H)),
                  full((3, H)), full((GB, A, A * A))],
        out_specs=out_specs,
        out_shape=out_shapes,
    )(t2, atom_types, frac_coords, l_polar, tfreq, sfreq,
      W_ts, b_ts.reshape(1, H), W_ne, b_ne.reshape(1, H),
      msg_W1, msg_b1.reshape(L, 1, H), msg_W2, msg_b2.reshape(L, 1, H),
      agg_W1, agg_b1.reshape(L, 1, H), agg_W2, agg_b2.reshape(L, 1, H),
      ln_g.reshape(L, 1, H), ln_b.reshape(L, 1, H),
      fln_g.reshape(1, H), fln_b.reshape(1, H),
      W_tr, b_tr.reshape(1, TYPE_DIM), W_lp, W_fc, sred)


# speedup vs baseline: 1.3519x; 1.1660x over previous
import math

import jax
import jax.numpy as jnp
from jax.experimental import pallas as pl

G = 512
A = 16
N = G * A
TYPE_DIM = 100
TIME_DIM = 128
H = 128
L = 4
NFREQ = 10
GB = 32


def _silu(x):
    return x * (0.5 * jnp.tanh(0.5 * x) + 0.5)


def _ln(x, g, b):
    m = jnp.mean(x, axis=-1, keepdims=True)
    xc = x - m
    v = jnp.mean(xc * xc, axis=-1, keepdims=True)
    return xc * jax.lax.rsqrt(v + 1e-5) * g + b


def _dotT(x, w):
    return jax.lax.dot_general(
        x, w, (((1,), (1,)), ((), ())), preferred_element_type=jnp.float32)


def _gnn_block(t_ref, at_ref, fc_ref, lp_ref, tfreq_ref, sfreq_ref,
               Wts_ref, bts_ref, Wne_ref, bne_ref,
               mW1_ref, mb1_ref, mW2_ref, mb2_ref,
               aW1_ref, ab1_ref, aW2_ref, ab2_ref,
               lng_ref, lnb_ref, flng_ref, flnb_ref,
               Wtr_ref, btr_ref, Wlp_ref, Wfc_ref,
               sred_ref, pat_ref,
               type_out, lpp_out, fcp_out):
    nb = GB * A
    E = GB * A * A
    inv_a = 1.0 / A
    F3 = 3 * NFREQ

    temb = _dotT(at_ref[...], Wts_ref[...]) + bts_ref[...]
    targ = t_ref[...] * tfreq_ref[...]
    temb_t = jnp.concatenate([jnp.sin(targ), jnp.cos(targ)], axis=1)
    Wne = Wne_ref[...]
    tproj = _dotT(temb_t, Wne[:, H:])
    nf = _dotT(temb, Wne[:, :H]) + bne_ref[...]
    nf = (nf.reshape(GB, A, H) + tproj[:, None, :]).reshape(nb, H)

    narg = jax.lax.dot_general(
        fc_ref[...], sfreq_ref[...], (((1,), (0,)), ((), ())),
        preferred_element_type=jnp.float32)
    ns = jnp.sin(narg)
    nc = jnp.cos(narg)
    nsj = ns.reshape(GB, 1, A, F3)
    ncj = nc.reshape(GB, 1, A, F3)
    nsi = ns.reshape(GB, A, 1, F3)
    nci = nc.reshape(GB, A, 1, F3)
    fsin = (nsj * nci - ncj * nsi).reshape(E, F3)
    fcos = (ncj * nci + nsj * nsi).reshape(E, F3)
    fsc = jnp.concatenate([fsin, fcos], axis=1)
    lp = lp_ref[...]

    for l in range(L):
        w1 = mW1_ref[l]
        nfn = _ln(nf, lng_ref[l], lnb_ref[l])
        asrc = _dotT(nfn, w1[:, :H])
        bdst = _dotT(nfn, w1[:, H:2 * H])
        cef = _dotT(fsc, w1[:, 2 * H:2 * H + 2 * F3])
        lpp = _dotT(lp, w1[:, 2 * H + 2 * F3:]) + mb1_ref[l]
        asrc = asrc.reshape(GB, A, H) + lpp[:, None, :]
        ab = jnp.concatenate([asrc, bdst.reshape(GB, A, H)], axis=1)
        pre = cef + jax.lax.dot_general(
            pat_ref[...], ab, (((2,), (1,)), ((0,), (0,))),
            preferred_element_type=jnp.float32).reshape(E, H)
        h = _dotT(_silu(pre).reshape(E, H), mW2_ref[l]) + mb2_ref[l]
        mij = _silu(h)
        msg = jax.lax.dot_general(
            sred_ref[...], mij.reshape(GB, A * A, H),
            (((2,), (1,)), ((0,), (0,))),
            preferred_element_type=jnp.float32).reshape(nb, H)
        aw1 = aW1_ref[l]
        agg = _dotT(nf, aw1[:, :H]) + _dotT(msg, aw1[:, H:]) + ab1_ref[l]
        agg = _silu(_dotT(_silu(agg), aW2_ref[l]) + ab2_ref[l])
        nf = nf + agg

    nff = _ln(nf, flng_ref[...], flnb_ref[...])
    gf = jnp.sum(nff.reshape(GB, A, H), axis=1) * inv_a
    type_out[...] = _dotT(nff, Wtr_ref[...]) + btr_ref[...]
    lpp_out[...] = _dotT(gf, Wlp_ref[...])
    fcp_out[...] = _dotT(nff, Wfc_ref[...])


def kernel(t, num_atoms, atom_types, frac_coords, l_polar, node2graph,
           W_ts, b_ts, W_ne, b_ne, msg_W1, msg_b1, msg_W2, msg_b2,
           agg_W1, agg_b1, agg_W2, agg_b2, ln_g, ln_b, fln_g, fln_b,
           W_tr, b_tr, W_lp, W_fc):
    f32 = jnp.float32
    half = TIME_DIM // 2
    msg_in = 2 * H + 6 * NFREQ + 6
    tfreq = jnp.exp(
        jnp.arange(half, dtype=f32) * (-(math.log(10000.0) / (half - 1)))
    ).reshape(1, half)
    sfreq = jnp.kron(
        jnp.eye(3, dtype=f32),
        (2.0 * math.pi * jnp.arange(NFREQ, dtype=f32)).reshape(1, NFREQ))
    t2 = t.reshape(G, 1)
    sred = jnp.broadcast_to(
        jnp.kron(jnp.eye(A, dtype=f32), jnp.full((1, A), 1.0 / A, f32)),
        (GB, A, A * A))
    pat = jnp.broadcast_to(
        jnp.concatenate([jnp.kron(jnp.eye(A, dtype=f32), jnp.ones((A, 1), f32)),
                         jnp.kron(jnp.ones((A, 1), f32), jnp.eye(A, dtype=f32))],
                        axis=1),
        (GB, A * A, 2 * A))

    nb = GB * A

    def full(shape):
        return pl.BlockSpec(shape, lambda i: tuple(0 for _ in shape))

    def node_bs(d):
        return pl.BlockSpec((nb, d), lambda i: (i, 0))

    def graph_bs(d):
        return pl.BlockSpec((GB, d), lambda i: (i, 0))

    out_shapes = (
        jax.ShapeDtypeStruct((N, TYPE_DIM), f32),
        jax.ShapeDtypeStruct((G, 6), f32),
        jax.ShapeDtypeStruct((N, 3), f32),
    )
    out_specs = (node_bs(TYPE_DIM), graph_bs(6), node_bs(3))

    return pl.pallas_call(
        _gnn_block,
        grid=(G // GB,),
        in_specs=[graph_bs(1), node_bs(TYPE_DIM), node_bs(3), graph_bs(6),
                  full((1, half)), full((3, 3 * NFREQ)),
                  full((H, TYPE_DIM)), full((1, H)),
                  full((H, H + TIME_DIM)), full((1, H)),
                  full((L, H, msg_in)), full((L, 1, H)),
                  full((L, H, H)), full((L, 1, H)),
                  full((L, H, 2 * H)), full((L, 1, H)),
                  full((L, H, H)), full((L, 1, H)),
                  full((L, 1, H)), full((L, 1, H)), full((1, H)),
                  full((1, H)),
                  full((TYPE_DIM, H)), full((1, TYPE_DIM)), full((6, H)),
                  full((3, H)), full((GB, A, A * A)),
                  full((GB, A * A, 2 * A))],
        out_specs=out_specs,
        out_shape=out_shapes,
    )(t2, atom_types, frac_coords, l_polar, tfreq, sfreq,
      W_ts, b_ts.reshape(1, H), W_ne, b_ne.reshape(1, H),
      msg_W1, msg_b1.reshape(L, 1, H), msg_W2, msg_b2.reshape(L, 1, H),
      agg_W1, agg_b1.reshape(L, 1, H), agg_W2, agg_b2.reshape(L, 1, H),
      ln_g.reshape(L, 1, H), ln_b.reshape(L, 1, H),
      fln_g.reshape(1, H), fln_b.reshape(1, H),
      W_tr, b_tr.reshape(1, TYPE_DIM), W_lp, W_fc, sred, pat)


# speedup vs baseline: 1.4679x; 1.0858x over previous
import math

import jax
import jax.numpy as jnp
from jax.experimental import pallas as pl

G = 512
A = 16
N = G * A
TYPE_DIM = 100
TIME_DIM = 128
H = 128
L = 4
NFREQ = 10
GB = 32


def _silu(x):
    return x * (0.5 * jnp.tanh(0.5 * x) + 0.5)


def _silu16(x):
    y = jnp.bfloat16(0.5) * x
    return y * jnp.tanh(y) + y


def _ln(x, g, b):
    m = jnp.mean(x, axis=-1, keepdims=True)
    xc = x - m
    v = jnp.mean(xc * xc, axis=-1, keepdims=True)
    return xc * jax.lax.rsqrt(v + 1e-5) * g + b


def _dotT(x, w):
    return jax.lax.dot_general(
        x, w, (((1,), (1,)), ((), ())), preferred_element_type=jnp.float32)


def _gnn_block(t_ref, at_ref, fc_ref, lp_ref, tfreq_ref, sfreq_ref,
               Wts_ref, bts_ref, Wne_ref, bne_ref,
               mW1_ref, mb1_ref, mW2_ref, mb2_ref,
               aW1_ref, ab1_ref, aW2_ref, ab2_ref,
               lng_ref, lnb_ref, flng_ref, flnb_ref,
               Wtr_ref, btr_ref, Wlp_ref, Wfc_ref,
               sred_ref, pat_ref,
               type_out, lpp_out, fcp_out):
    nb = GB * A
    E = GB * A * A
    inv_a = 1.0 / A
    F3 = 3 * NFREQ

    temb = _dotT(at_ref[...], Wts_ref[...]) + bts_ref[...]
    targ = t_ref[...] * tfreq_ref[...]
    temb_t = jnp.concatenate([jnp.sin(targ), jnp.cos(targ)], axis=1)
    Wne = Wne_ref[...]
    tproj = _dotT(temb_t, Wne[:, H:])
    nf = _dotT(temb, Wne[:, :H]) + bne_ref[...]
    nf = (nf.reshape(GB, A, H) + tproj[:, None, :]).reshape(nb, H)

    narg = jax.lax.dot_general(
        fc_ref[...], sfreq_ref[...], (((1,), (0,)), ((), ())),
        preferred_element_type=jnp.float32)
    ns = jnp.sin(narg)
    nc = jnp.cos(narg)
    nsj = ns.reshape(GB, 1, A, F3)
    ncj = nc.reshape(GB, 1, A, F3)
    nsi = ns.reshape(GB, A, 1, F3)
    nci = nc.reshape(GB, A, 1, F3)
    fsin = (nsj * nci - ncj * nsi).reshape(E, F3)
    fcos = (ncj * nci + nsj * nsi).reshape(E, F3)
    fsc = jnp.concatenate([fsin, fcos], axis=1).astype(jnp.bfloat16)
    lp = lp_ref[...]

    for l in range(L):
        w1 = mW1_ref[l]
        nfn = _ln(nf, lng_ref[l], lnb_ref[l])
        asrc = _dotT(nfn, w1[:, :H])
        bdst = _dotT(nfn, w1[:, H:2 * H])
        cef = jax.lax.dot_general(
            fsc, w1[:, 2 * H:2 * H + 2 * F3].astype(jnp.bfloat16),
            (((1,), (1,)), ((), ())),
            preferred_element_type=jnp.float32)
        lpp = _dotT(lp, w1[:, 2 * H + 2 * F3:]) + mb1_ref[l]
        asrc = asrc.reshape(GB, A, H) + lpp[:, None, :]
        ab = jnp.concatenate(
            [asrc, bdst.reshape(GB, A, H)], axis=1).astype(jnp.bfloat16)
        pre = cef + jax.lax.dot_general(
            pat_ref[...], ab, (((2,), (1,)), ((0,), (0,))),
            preferred_element_type=jnp.float32).reshape(E, H)
        h = jax.lax.dot_general(
            _silu16(pre.astype(jnp.bfloat16)).reshape(E, H),
            mW2_ref[l].astype(jnp.bfloat16),
            (((1,), (1,)), ((), ())),
            preferred_element_type=jnp.float32) + mb2_ref[l]
        mij = _silu16(h.astype(jnp.bfloat16))
        msg = jax.lax.dot_general(
            sred_ref[...], mij.reshape(GB, A * A, H),
            (((2,), (1,)), ((0,), (0,))),
            preferred_element_type=jnp.float32).reshape(nb, H)
        aw1 = aW1_ref[l]
        agg = _dotT(nf, aw1[:, :H]) + _dotT(msg, aw1[:, H:]) + ab1_ref[l]
        agg = _silu(_dotT(_silu(agg), aW2_ref[l]) + ab2_ref[l])
        nf = nf + agg

    nff = _ln(nf, flng_ref[...], flnb_ref[...])
    gf = jnp.sum(nff.reshape(GB, A, H), axis=1) * inv_a
    type_out[...] = _dotT(nff, Wtr_ref[...]) + btr_ref[...]
    lpp_out[...] = _dotT(gf, Wlp_ref[...])
    fcp_out[...] = _dotT(nff, Wfc_ref[...])


def kernel(t, num_atoms, atom_types, frac_coords, l_polar, node2graph,
           W_ts, b_ts, W_ne, b_ne, msg_W1, msg_b1, msg_W2, msg_b2,
           agg_W1, agg_b1, agg_W2, agg_b2, ln_g, ln_b, fln_g, fln_b,
           W_tr, b_tr, W_lp, W_fc):
    f32 = jnp.float32
    half = TIME_DIM // 2
    msg_in = 2 * H + 6 * NFREQ + 6
    tfreq = jnp.exp(
        jnp.arange(half, dtype=f32) * (-(math.log(10000.0) / (half - 1)))
    ).reshape(1, half)
    sfreq = jnp.kron(
        jnp.eye(3, dtype=f32),
        (2.0 * math.pi * jnp.arange(NFREQ, dtype=f32)).reshape(1, NFREQ))
    t2 = t.reshape(G, 1)
    sred = jnp.broadcast_to(
        jnp.kron(jnp.eye(A, dtype=jnp.bfloat16),
                 jnp.full((1, A), 1.0 / A, jnp.bfloat16)),
        (GB, A, A * A))
    bf16 = jnp.bfloat16
    pat = jnp.broadcast_to(
        jnp.concatenate([jnp.kron(jnp.eye(A, dtype=bf16), jnp.ones((A, 1), bf16)),
                         jnp.kron(jnp.ones((A, 1), bf16), jnp.eye(A, dtype=bf16))],
                        axis=1),
        (GB, A * A, 2 * A))

    nb = GB * A

    def full(shape):
        return pl.BlockSpec(shape, lambda i: tuple(0 for _ in shape))

    def node_bs(d):
        return pl.BlockSpec((nb, d), lambda i: (i, 0))

    def graph_bs(d):
        return pl.BlockSpec((GB, d), lambda i: (i, 0))

    out_shapes = (
        jax.ShapeDtypeStruct((N, TYPE_DIM), f32),
        jax.ShapeDtypeStruct((G, 6), f32),
        jax.ShapeDtypeStruct((N, 3), f32),
    )
    out_specs = (node_bs(TYPE_DIM), graph_bs(6), node_bs(3))

    return pl.pallas_call(
        _gnn_block,
        grid=(G // GB,),
        in_specs=[graph_bs(1), node_bs(TYPE_DIM), node_bs(3), graph_bs(6),
                  full((1, half)), full((3, 3 * NFREQ)),
                  full((H, TYPE_DIM)), full((1, H)),
                  full((H, H + TIME_DIM)), full((1, H)),
                  full((L, H, msg_in)), full((L, 1, H)),
                  full((L, H, H)), full((L, 1, H)),
                  full((L, H, 2 * H)), full((L, 1, H)),
                  full((L, H, H)), full((L, 1, H)),
                  full((L, 1, H)), full((L, 1, H)), full((1, H)),
                  full((1, H)),
                  full((TYPE_DIM, H)), full((1, TYPE_DIM)), full((6, H)),
                  full((3, H)), full((GB, A, A * A)),
                  full((GB, A * A, 2 * A))],
        out_specs=out_specs,
        out_shape=out_shapes,
    )(t2, atom_types, frac_coords, l_polar, tfreq, sfreq,
      W_ts, b_ts.reshape(1, H), W_ne, b_ne.reshape(1, H),
      msg_W1, msg_b1.reshape(L, 1, H), msg_W2, msg_b2.reshape(L, 1, H),
      agg_W1, agg_b1.reshape(L, 1, H), agg_W2, agg_b2.reshape(L, 1, H),
      ln_g.reshape(L, 1, H), ln_b.reshape(L, 1, H),
      fln_g.reshape(1, H), fln_b.reshape(1, H),
      W_tr, b_tr.reshape(1, TYPE_DIM), W_lp, W_fc, sred, pat)


# speedup vs baseline: 1.5209x; 1.0361x over previous
import math

import jax
import jax.numpy as jnp
from jax.experimental import pallas as pl

G = 512
A = 16
N = G * A
TYPE_DIM = 100
TIME_DIM = 128
H = 128
L = 4
NFREQ = 10
GB = 32


def _silu(x):
    return x * (0.5 * jnp.tanh(0.5 * x) + 0.5)


def _silu16(x):
    y = jnp.bfloat16(0.5) * x
    return y * jnp.tanh(y) + y


def _ln(x, g, b):
    m = jnp.mean(x, axis=-1, keepdims=True)
    xc = x - m
    v = jnp.mean(xc * xc, axis=-1, keepdims=True)
    return xc * jax.lax.rsqrt(v + 1e-5) * g + b


def _dotT(x, w):
    return jax.lax.dot_general(
        x, w, (((1,), (1,)), ((), ())), preferred_element_type=jnp.float32)


def _gnn_block(t_ref, at_ref, fc_ref, lp_ref, tfreq_ref, sfreq_ref,
               Wts_ref, bts_ref, Wne_ref, bne_ref,
               mW1_ref, mb1_ref, mW2_ref, mb2_ref,
               aW1_ref, ab1_ref, aW2_ref, ab2_ref,
               lng_ref, lnb_ref, flng_ref, flnb_ref,
               Wtr_ref, btr_ref, Wlp_ref, Wfc_ref,
               sred_ref, pat_ref,
               type_out, lpp_out, fcp_out):
    nb = GB * A
    E = GB * A * A
    inv_a = 1.0 / A
    F3 = 3 * NFREQ

    temb = _dotT(at_ref[...], Wts_ref[...]) + bts_ref[...]
    targ = t_ref[...] * tfreq_ref[...]
    temb_t = jnp.concatenate([jnp.sin(targ), jnp.cos(targ)], axis=1)
    Wne = Wne_ref[...]
    tproj = _dotT(temb_t, Wne[:, H:])
    nf = _dotT(temb, Wne[:, :H]) + bne_ref[...]
    nf = (nf.reshape(GB, A, H) + tproj[:, None, :]).reshape(nb, H)

    narg = jax.lax.dot_general(
        fc_ref[...], sfreq_ref[...], (((1,), (0,)), ((), ())),
        preferred_element_type=jnp.float32)
    ns = jnp.sin(narg)
    nc = jnp.cos(narg)
    nsj = ns.reshape(GB, 1, A, F3)
    ncj = nc.reshape(GB, 1, A, F3)
    nsi = ns.reshape(GB, A, 1, F3)
    nci = nc.reshape(GB, A, 1, F3)
    fsin = (nsj * nci - ncj * nsi).reshape(E, F3)
    fcos = (ncj * nci + nsj * nsi).reshape(E, F3)
    efeat = jnp.concatenate(
        [jnp.concatenate([fsin, fcos], axis=1).astype(jnp.bfloat16),
         pat_ref[...].reshape(E, 2 * A)], axis=1).reshape(GB, A * A, 2 * F3 + 2 * A)
    lp = lp_ref[...]

    for l in range(L):
        w1 = mW1_ref[l]
        nfn = _ln(nf, lng_ref[l], lnb_ref[l])
        asrc = _dotT(nfn, w1[:, :H])
        bdst = _dotT(nfn, w1[:, H:2 * H])
        lpp = _dotT(lp, w1[:, 2 * H + 2 * F3:]) + mb1_ref[l]
        asrc = asrc.reshape(GB, A, H) + lpp[:, None, :]
        rhs = jnp.concatenate(
            [jnp.broadcast_to(
                w1[:, 2 * H:2 * H + 2 * F3].astype(jnp.bfloat16).T[None],
                (GB, 2 * F3, H)),
             asrc.astype(jnp.bfloat16),
             bdst.reshape(GB, A, H).astype(jnp.bfloat16)], axis=1)
        pre = jax.lax.dot_general(
            efeat, rhs, (((2,), (1,)), ((0,), (0,))),
            preferred_element_type=jnp.float32).reshape(E, H)
        h = jax.lax.dot_general(
            _silu16(pre.astype(jnp.bfloat16)).reshape(E, H),
            mW2_ref[l].astype(jnp.bfloat16),
            (((1,), (1,)), ((), ())),
            preferred_element_type=jnp.float32) + mb2_ref[l]
        mij = _silu16(h.astype(jnp.bfloat16))
        msg = jax.lax.dot_general(
            sred_ref[...], mij.reshape(GB, A * A, H),
            (((2,), (1,)), ((0,), (0,))),
            preferred_element_type=jnp.float32).reshape(nb, H)
        aw1 = aW1_ref[l]
        agg = _dotT(nf, aw1[:, :H]) + _dotT(msg, aw1[:, H:]) + ab1_ref[l]
        agg = _silu(_dotT(_silu(agg), aW2_ref[l]) + ab2_ref[l])
        nf = nf + agg

    nff = _ln(nf, flng_ref[...], flnb_ref[...])
    gf = jnp.sum(nff.reshape(GB, A, H), axis=1) * inv_a
    type_out[...] = _dotT(nff, Wtr_ref[...]) + btr_ref[...]
    lpp_out[...] = _dotT(gf, Wlp_ref[...])
    fcp_out[...] = _dotT(nff, Wfc_ref[...])


def kernel(t, num_atoms, atom_types, frac_coords, l_polar, node2graph,
           W_ts, b_ts, W_ne, b_ne, msg_W1, msg_b1, msg_W2, msg_b2,
           agg_W1, agg_b1, agg_W2, agg_b2, ln_g, ln_b, fln_g, fln_b,
           W_tr, b_tr, W_lp, W_fc):
    f32 = jnp.float32
    half = TIME_DIM // 2
    msg_in = 2 * H + 6 * NFREQ + 6
    tfreq = jnp.exp(
        jnp.arange(half, dtype=f32) * (-(math.log(10000.0) / (half - 1)))
    ).reshape(1, half)
    sfreq = jnp.kron(
        jnp.eye(3, dtype=f32),
        (2.0 * math.pi * jnp.arange(NFREQ, dtype=f32)).reshape(1, NFREQ))
    t2 = t.reshape(G, 1)
    sred = jnp.broadcast_to(
        jnp.kron(jnp.eye(A, dtype=jnp.bfloat16),
                 jnp.full((1, A), 1.0 / A, jnp.bfloat16)),
        (GB, A, A * A))
    bf16 = jnp.bfloat16
    pat = jnp.broadcast_to(
        jnp.concatenate([jnp.kron(jnp.eye(A, dtype=bf16), jnp.ones((A, 1), bf16)),
                         jnp.kron(jnp.ones((A, 1), bf16), jnp.eye(A, dtype=bf16))],
                        axis=1),
        (GB, A * A, 2 * A))

    nb = GB * A

    def full(shape):
        return pl.BlockSpec(shape, lambda i: tuple(0 for _ in shape))

    def node_bs(d):
        return pl.BlockSpec((nb, d), lambda i: (i, 0))

    def graph_bs(d):
        return pl.BlockSpec((GB, d), lambda i: (i, 0))

    out_shapes = (
        jax.ShapeDtypeStruct((N, TYPE_DIM), f32),
        jax.ShapeDtypeStruct((G, 6), f32),
        jax.ShapeDtypeStruct((N, 3), f32),
    )
    out_specs = (node_bs(TYPE_DIM), graph_bs(6), node_bs(3))

    return pl.pallas_call(
        _gnn_block,
        grid=(G // GB,),
        in_specs=[graph_bs(1), node_bs(TYPE_DIM), node_bs(3), graph_bs(6),
                  full((1, half)), full((3, 3 * NFREQ)),
                  full((H, TYPE_DIM)), full((1, H)),
                  full((H, H + TIME_DIM)), full((1, H)),
                  full((L, H, msg_in)), full((L, 1, H)),
                  full((L, H, H)), full((L, 1, H)),
                  full((L, H, 2 * H)), full((L, 1, H)),
                  full((L, H, H)), full((L, 1, H)),
                  full((L, 1, H)), full((L, 1, H)), full((1, H)),
                  full((1, H)),
                  full((TYPE_DIM, H)), full((1, TYPE_DIM)), full((6, H)),
                  full((3, H)), full((GB, A, A * A)),
                  full((GB, A * A, 2 * A))],
        out_specs=out_specs,
        out_shape=out_shapes,
    )(t2, atom_types, frac_coords, l_polar, tfreq, sfreq,
      W_ts, b_ts.reshape(1, H), W_ne, b_ne.reshape(1, H),
      msg_W1, msg_b1.reshape(L, 1, H), msg_W2, msg_b2.reshape(L, 1, H),
      agg_W1, agg_b1.reshape(L, 1, H), agg_W2, agg_b2.reshape(L, 1, H),
      ln_g.reshape(L, 1, H), ln_b.reshape(L, 1, H),
      fln_g.reshape(1, H), fln_b.reshape(1, H),
      W_tr, b_tr.reshape(1, TYPE_DIM), W_lp, W_fc, sred, pat)


# speedup vs baseline: 1.6768x; 1.1025x over previous
import math

import jax
import jax.numpy as jnp
from jax.experimental import pallas as pl

G = 512
A = 16
N = G * A
TYPE_DIM = 100
TIME_DIM = 128
H = 128
L = 4
NFREQ = 10
GB = 64


def _silu(x):
    return x * (0.5 * jnp.tanh(0.5 * x) + 0.5)


def _silu16(x):
    y = jnp.bfloat16(0.5) * x
    return y * jnp.tanh(y) + y


def _ln(x, g, b):
    m = jnp.mean(x, axis=-1, keepdims=True)
    xc = x - m
    v = jnp.mean(xc * xc, axis=-1, keepdims=True)
    return xc * jax.lax.rsqrt(v + 1e-5) * g + b


def _dotT(x, w):
    return jax.lax.dot_general(
        x, w, (((1,), (1,)), ((), ())), preferred_element_type=jnp.float32)


def _gnn_block(t_ref, at_ref, fc_ref, lp_ref, tfreq_ref, sfreq_ref,
               Wts_ref, bts_ref, Wne_ref, bne_ref,
               mW1_ref, mb1_ref, mW2_ref, mb2_ref,
               aW1_ref, ab1_ref, aW2_ref, ab2_ref,
               lng_ref, lnb_ref, flng_ref, flnb_ref,
               Wtr_ref, btr_ref, Wlp_ref, Wfc_ref,
               sred_ref, pat_ref,
               type_out, lpp_out, fcp_out):
    nb = GB * A
    E = GB * A * A
    inv_a = 1.0 / A
    F3 = 3 * NFREQ

    temb = _dotT(at_ref[...], Wts_ref[...]) + bts_ref[...]
    targ = t_ref[...] * tfreq_ref[...]
    temb_t = jnp.concatenate([jnp.sin(targ), jnp.cos(targ)], axis=1)
    Wne = Wne_ref[...]
    tproj = _dotT(temb_t, Wne[:, H:])
    nf = _dotT(temb, Wne[:, :H]) + bne_ref[...]
    nf = (nf.reshape(GB, A, H) + tproj[:, None, :]).reshape(nb, H)

    narg = jax.lax.dot_general(
        fc_ref[...], sfreq_ref[...], (((1,), (0,)), ((), ())),
        preferred_element_type=jnp.float32)
    ns = jnp.sin(narg)
    nc = jnp.cos(narg)
    nsj = ns.reshape(GB, 1, A, F3)
    ncj = nc.reshape(GB, 1, A, F3)
    nsi = ns.reshape(GB, A, 1, F3)
    nci = nc.reshape(GB, A, 1, F3)
    fsin = (nsj * nci - ncj * nsi).reshape(E, F3)
    fcos = (ncj * nci + nsj * nsi).reshape(E, F3)
    efeat = jnp.concatenate(
        [jnp.concatenate([fsin, fcos], axis=1).astype(jnp.bfloat16),
         pat_ref[...].reshape(E, 2 * A)], axis=1).reshape(GB, A * A, 2 * F3 + 2 * A)
    lp = lp_ref[...]

    for l in range(L):
        w1 = mW1_ref[l]
        nfn = _ln(nf, lng_ref[l], lnb_ref[l])
        asrc = _dotT(nfn, w1[:, :H])
        bdst = _dotT(nfn, w1[:, H:2 * H])
        lpp = _dotT(lp, w1[:, 2 * H + 2 * F3:]) + mb1_ref[l]
        asrc = asrc.reshape(GB, A, H) + lpp[:, None, :]
        rhs = jnp.concatenate(
            [jnp.broadcast_to(
                w1[:, 2 * H:2 * H + 2 * F3].astype(jnp.bfloat16).T[None],
                (GB, 2 * F3, H)),
             asrc.astype(jnp.bfloat16),
             bdst.reshape(GB, A, H).astype(jnp.bfloat16)], axis=1)
        pre = jax.lax.dot_general(
            efeat, rhs, (((2,), (1,)), ((0,), (0,))),
            preferred_element_type=jnp.float32).reshape(E, H)
        h = jax.lax.dot_general(
            _silu16(pre.astype(jnp.bfloat16)).reshape(E, H),
            mW2_ref[l].astype(jnp.bfloat16),
            (((1,), (1,)), ((), ())),
            preferred_element_type=jnp.float32) + mb2_ref[l]
        mij = _silu16(h.astype(jnp.bfloat16))
        msg = jax.lax.dot_general(
            sred_ref[...], mij.reshape(GB, A * A, H),
            (((2,), (1,)), ((0,), (0,))),
            preferred_element_type=jnp.float32).reshape(nb, H)
        aw1 = aW1_ref[l]
        agg = _dotT(nf, aw1[:, :H]) + _dotT(msg, aw1[:, H:]) + ab1_ref[l]
        agg = _silu(_dotT(_silu(agg), aW2_ref[l]) + ab2_ref[l])
        nf = nf + agg

    nff = _ln(nf, flng_ref[...], flnb_ref[...])
    gf = jnp.sum(nff.reshape(GB, A, H), axis=1) * inv_a
    type_out[...] = _dotT(nff, Wtr_ref[...]) + btr_ref[...]
    lpp_out[...] = _dotT(gf, Wlp_ref[...])
    fcp_out[...] = _dotT(nff, Wfc_ref[...])


def kernel(t, num_atoms, atom_types, frac_coords, l_polar, node2graph,
           W_ts, b_ts, W_ne, b_ne, msg_W1, msg_b1, msg_W2, msg_b2,
           agg_W1, agg_b1, agg_W2, agg_b2, ln_g, ln_b, fln_g, fln_b,
           W_tr, b_tr, W_lp, W_fc):
    f32 = jnp.float32
    half = TIME_DIM // 2
    msg_in = 2 * H + 6 * NFREQ + 6
    tfreq = jnp.exp(
        jnp.arange(half, dtype=f32) * (-(math.log(10000.0) / (half - 1)))
    ).reshape(1, half)
    sfreq = jnp.kron(
        jnp.eye(3, dtype=f32),
        (2.0 * math.pi * jnp.arange(NFREQ, dtype=f32)).reshape(1, NFREQ))
    t2 = t.reshape(G, 1)
    sred = jnp.broadcast_to(
        jnp.kron(jnp.eye(A, dtype=jnp.bfloat16),
                 jnp.full((1, A), 1.0 / A, jnp.bfloat16)),
        (GB, A, A * A))
    bf16 = jnp.bfloat16
    pat = jnp.broadcast_to(
        jnp.concatenate([jnp.kron(jnp.eye(A, dtype=bf16), jnp.ones((A, 1), bf16)),
                         jnp.kron(jnp.ones((A, 1), bf16), jnp.eye(A, dtype=bf16))],
                        axis=1),
        (GB, A * A, 2 * A))

    nb = GB * A

    def full(shape):
        return pl.BlockSpec(shape, lambda i: tuple(0 for _ in shape))

    def node_bs(d):
        return pl.BlockSpec((nb, d), lambda i: (i, 0))

    def graph_bs(d):
        return pl.BlockSpec((GB, d), lambda i: (i, 0))

    out_shapes = (
        jax.ShapeDtypeStruct((N, TYPE_DIM), f32),
        jax.ShapeDtypeStruct((G, 6), f32),
        jax.ShapeDtypeStruct((N, 3), f32),
    )
    out_specs = (node_bs(TYPE_DIM), graph_bs(6), node_bs(3))

    return pl.pallas_call(
        _gnn_block,
        grid=(G // GB,),
        in_specs=[graph_bs(1), node_bs(TYPE_DIM), node_bs(3), graph_bs(6),
                  full((1, half)), full((3, 3 * NFREQ)),
                  full((H, TYPE_DIM)), full((1, H)),
                  full((H, H + TIME_DIM)), full((1, H)),
                  full((L, H, msg_in)), full((L, 1, H)),
                  full((L, H, H)), full((L, 1, H)),
                  full((L, H, 2 * H)), full((L, 1, H)),
                  full((L, H, H)), full((L, 1, H)),
                  full((L, 1, H)), full((L, 1, H)), full((1, H)),
                  full((1, H)),
                  full((TYPE_DIM, H)), full((1, TYPE_DIM)), full((6, H)),
                  full((3, H)), full((GB, A, A * A)),
                  full((GB, A * A, 2 * A))],
        out_specs=out_specs,
        out_shape=out_shapes,
    )(t2, atom_types, frac_coords, l_polar, tfreq, sfreq,
      W_ts, b_ts.reshape(1, H), W_ne, b_ne.reshape(1, H),
      msg_W1, msg_b1.reshape(L, 1, H), msg_W2, msg_b2.reshape(L, 1, H),
      agg_W1, agg_b1.reshape(L, 1, H), agg_W2, agg_b2.reshape(L, 1, H),
      ln_g.reshape(L, 1, H), ln_b.reshape(L, 1, H),
      fln_g.reshape(1, H), fln_b.reshape(1, H),
      W_tr, b_tr.reshape(1, TYPE_DIM), W_lp, W_fc, sred, pat)
